# baseline (speedup 1.0000x reference)
import functools

import jax
import jax.numpy as jnp
from jax.experimental import pallas as pl
from jax.experimental.pallas import tpu as pltpu

_E = 8
_LANES = 128
_NEG_INF = float("-inf")


def _moe_body(x_ref, gwt_ref, a_ref, bpad_ref, out_ref, xw_ref):
    e = pl.program_id(0)

    @pl.when(e == 0)
    def _routing_and_stage():
        xt = x_ref[...]
        logits = jnp.dot(xt, gwt_ref[...],
                         preferred_element_type=jnp.float32)
        lane = jax.lax.broadcasted_iota(jnp.int32, logits.shape, 1)
        logits = jnp.where(lane < _E, logits, _NEG_INF)
        m1 = jnp.max(logits, axis=1, keepdims=True)
        idx0 = jnp.min(jnp.where(logits == m1, lane, _LANES), axis=1,
                       keepdims=True)
        logits2 = jnp.where(lane == idx0, _NEG_INF, logits)
        m2 = jnp.max(logits2, axis=1, keepdims=True)
        idx1 = jnp.min(jnp.where(logits2 == m2, lane, _LANES), axis=1,
                       keepdims=True)
        s = jnp.exp(m2 - m1)
        w0 = 1.0 / (1.0 + s)
        w1 = 1.0 - w0
        dense_w = (jnp.where(lane == idx0, w0, 0.0)
                   + jnp.where(lane == idx1, w1, 0.0))
        out_ref[...] = jnp.dot(dense_w, bpad_ref[...],
                               preferred_element_type=jnp.float32)
        for ee in range(_E):
            xw_ref[ee] = (dense_w[:, ee:ee + 1] * xt).astype(jnp.bfloat16)

    out_ref[...] += jnp.dot(xw_ref[e], a_ref[0].astype(jnp.bfloat16),
                            preferred_element_type=jnp.float32)


@functools.partial(jax.jit, static_argnames=())
def kernel(inputs, gate_w, expert_A, expert_b):
    batch_shape = inputs.shape[:-1]
    d = inputs.shape[-1]
    x = inputs.reshape(-1, d)
    t = x.shape[0]

    gwt = jnp.zeros((d, _LANES), dtype=gate_w.dtype).at[:, :_E].set(gate_w.T)
    bpad = jnp.zeros((_LANES, d), dtype=expert_b.dtype).at[:_E].set(expert_b)

    out = pl.pallas_call(
        _moe_body,
        grid=(_E,),
        in_specs=[
            pl.BlockSpec((t, d), lambda e: (0, 0)),
            pl.BlockSpec((d, _LANES), lambda e: (0, 0)),
            pl.BlockSpec((1, d, d), lambda e: (e, 0, 0)),
            pl.BlockSpec((_LANES, d), lambda e: (0, 0)),
        ],
        out_specs=pl.BlockSpec((t, d), lambda e: (0, 0)),
        out_shape=jax.ShapeDtypeStruct((t, d), jnp.float32),
        scratch_shapes=[
            pltpu.VMEM((_E, t, d), jnp.bfloat16),
        ],
    )(x, gwt, expert_A, bpad)
    return out.reshape(*batch_shape, d)
